# baseline (speedup 1.0000x reference)
import functools

import jax
import jax.numpy as jnp
from jax import lax
from jax.experimental import pallas as pl
from jax.experimental.pallas import tpu as pltpu
from jax.experimental.pallas import tpu_sc as plsc

B = 4096
D_IN = 256
D_BLOCK = 512
N_EXPERTS = 8
TOP_K = 2
D_OUT = 10
D_OUT_PAD = 128

BT_G = 256
P = 2 * B + N_EXPERTS * BT_G
NBLK = P // BT_G
CH = 256


def _router_body(x_ref, wg_ref, bg_ref, p1_ref, p2_ref, g1_ref, g2_ref,
                 be_ref, r1_ref, r2_ref):
    x = x_ref[...]
    logits = jnp.dot(x, wg_ref[...], preferred_element_type=jnp.float32)
    logits = logits + bg_ref[...]

    eids = lax.broadcasted_iota(jnp.int32, logits.shape, 1)
    m1 = jnp.max(logits, axis=-1, keepdims=True)
    i1 = jnp.min(jnp.where(logits == m1, eids, N_EXPERTS), axis=-1,
                 keepdims=True)
    masked = jnp.where(eids == i1, jnp.float32(-jnp.inf), logits)
    m2 = jnp.max(masked, axis=-1, keepdims=True)
    i2 = jnp.min(jnp.where(masked == m2, eids, N_EXPERTS), axis=-1,
                 keepdims=True)
    e2 = jnp.exp(m2 - m1)
    g1_ref[...] = 1.0 / (1.0 + e2)
    g2_ref[...] = e2 / (1.0 + e2)

    rows = lax.broadcasted_iota(jnp.int32, (CH, CH), 0)
    cols = lax.broadcasted_iota(jnp.int32, (CH, CH), 1)
    ls = (cols < rows).astype(jnp.float32)

    def scan_stream(i_sel, rank_ref, running):
        for c in range(B // CH):
            sl = slice(c * CH, (c + 1) * CH)
            oh = (i_sel[sl, :] == eids[:CH, :]).astype(jnp.float32)
            ex = jnp.dot(ls, oh, preferred_element_type=jnp.float32)
            rank_ref[sl, :] = ex + running
            running = running + jnp.sum(oh, axis=0, keepdims=True)
        return running

    running = jnp.zeros((1, N_EXPERTS), dtype=jnp.float32)
    running = scan_stream(i1, r1_ref, running)
    counts1 = running
    running = scan_stream(i2, r2_ref, running)
    counts = running

    pc = jnp.floor((counts + (BT_G - 1)) * (1.0 / BT_G)) * BT_G
    su = (lax.broadcasted_iota(jnp.int32, (N_EXPERTS, N_EXPERTS), 0)
          < lax.broadcasted_iota(jnp.int32, (N_EXPERTS, N_EXPERTS), 1))
    off = jnp.dot(pc, su.astype(jnp.float32),
                  preferred_element_type=jnp.float32)

    for c in range(B // CH):
        sl = slice(c * CH, (c + 1) * CH)
        oh1 = (i1[sl, :] == eids[:CH, :]).astype(jnp.float32)
        oh2 = (i2[sl, :] == eids[:CH, :]).astype(jnp.float32)
        pos1 = jnp.sum(oh1 * (r1_ref[sl, :] + off), axis=-1, keepdims=True)
        pos2 = jnp.sum(oh2 * (r2_ref[sl, :] + off), axis=-1, keepdims=True)
        p1_ref[sl, :] = pos1.astype(jnp.int32)
        p2_ref[sl, :] = pos2.astype(jnp.int32)

    bidx = lax.broadcasted_iota(jnp.int32, (64, N_EXPERTS), 0).astype(
        jnp.float32) * BT_G
    ends = off + pc
    be = jnp.sum((bidx >= ends).astype(jnp.float32), axis=-1, keepdims=True)
    be_ref[...] = jnp.minimum(be, N_EXPERTS - 1).astype(jnp.int32)


@jax.jit
def _router(x, Wg, bg2d):
    return pl.pallas_call(
        _router_body,
        grid=(1,),
        in_specs=[
            pl.BlockSpec((B, D_IN), lambda i: (0, 0)),
            pl.BlockSpec((D_IN, N_EXPERTS), lambda i: (0, 0)),
            pl.BlockSpec((1, N_EXPERTS), lambda i: (0, 0)),
        ],
        out_specs=[
            pl.BlockSpec((B, 1), lambda i: (0, 0)),
            pl.BlockSpec((B, 1), lambda i: (0, 0)),
            pl.BlockSpec((B, 1), lambda i: (0, 0)),
            pl.BlockSpec((B, 1), lambda i: (0, 0)),
            pl.BlockSpec((64, 1), lambda i: (0, 0)),
        ],
        out_shape=[
            jax.ShapeDtypeStruct((B, 1), jnp.int32),
            jax.ShapeDtypeStruct((B, 1), jnp.int32),
            jax.ShapeDtypeStruct((B, 1), jnp.float32),
            jax.ShapeDtypeStruct((B, 1), jnp.float32),
            jax.ShapeDtypeStruct((64, 1), jnp.int32),
        ],
        scratch_shapes=[
            pltpu.VMEM((B, N_EXPERTS), jnp.float32),
            pltpu.VMEM((B, N_EXPERTS), jnp.float32),
        ],
    )(x, Wg, bg2d)


def _gemm_body(be_ref, gx_ref, w1_ref, b1_ref, w2_ref, b2_ref, rows_ref):
    x = gx_ref[...]
    h = jnp.dot(x, w1_ref[0], preferred_element_type=jnp.float32)
    h = jnp.maximum(h + b1_ref[0], 0.0)
    o = jnp.dot(h, w2_ref[0], preferred_element_type=jnp.float32)
    rows_ref[...] = o + b2_ref[0]


@jax.jit
def _grouped_gemm(be, gx, W1, b1, W2p, b2p):
    grid_spec = pltpu.PrefetchScalarGridSpec(
        num_scalar_prefetch=1,
        grid=(NBLK,),
        in_specs=[
            pl.BlockSpec((BT_G, D_IN), lambda i, be: (i, 0)),
            pl.BlockSpec((1, D_IN, D_BLOCK), lambda i, be: (be[i], 0, 0)),
            pl.BlockSpec((1, 1, D_BLOCK), lambda i, be: (be[i], 0, 0)),
            pl.BlockSpec((1, D_BLOCK, D_OUT_PAD), lambda i, be: (be[i], 0, 0)),
            pl.BlockSpec((1, 1, D_OUT_PAD), lambda i, be: (be[i], 0, 0)),
        ],
        out_specs=pl.BlockSpec((BT_G, D_OUT_PAD), lambda i, be: (i, 0)),
    )
    return pl.pallas_call(
        _gemm_body,
        grid_spec=grid_spec,
        out_shape=jax.ShapeDtypeStruct((P, D_OUT_PAD), jnp.float32),
    )(be, gx, W1, b1, W2p, b2p)


_SC_INFO = plsc.get_sparse_core_info()
_NC, _NS = _SC_INFO.num_cores, _SC_INFO.num_subcores
_NW = _NC * _NS
TOK_W = B // _NW

_sc_mesh = plsc.VectorSubcoreMesh(core_axis_name="c", subcore_axis_name="s")


@functools.partial(
    pl.kernel, mesh=_sc_mesh,
    out_type=jax.ShapeDtypeStruct((P, D_IN), jnp.float32),
    scratch_types=[
        pltpu.VMEM((TOK_W, D_IN), jnp.float32),
        pltpu.VMEM((TOK_W,), jnp.int32),
        pltpu.VMEM((TOK_W,), jnp.int32),
        pltpu.SemaphoreType.DMA,
    ],
)
def _sc_scatter(x_hbm, p1_hbm, p2_hbm, gx_hbm, xv, p1v, p2v, sem):
    wid = lax.axis_index("s") * _NC + lax.axis_index("c")
    base = wid * TOK_W
    pltpu.sync_copy(x_hbm.at[pl.ds(base, TOK_W), :], xv)
    pltpu.sync_copy(p1_hbm.at[pl.ds(base, TOK_W)], p1v)
    pltpu.sync_copy(p2_hbm.at[pl.ds(base, TOK_W)], p2v)
    pltpu.async_copy(xv, gx_hbm.at[p1v], sem).wait()
    pltpu.async_copy(xv, gx_hbm.at[p2v], sem).wait()


@functools.partial(
    pl.kernel, mesh=_sc_mesh,
    out_type=[jax.ShapeDtypeStruct((B, D_OUT_PAD), jnp.float32),
              jax.ShapeDtypeStruct((B, D_OUT_PAD), jnp.float32)],
    scratch_types=[
        pltpu.VMEM((TOK_W,), jnp.int32),
        pltpu.VMEM((TOK_W,), jnp.int32),
        pltpu.VMEM((TOK_W, D_OUT_PAD), jnp.float32),
        pltpu.VMEM((TOK_W, D_OUT_PAD), jnp.float32),
        pltpu.SemaphoreType.DMA,
    ],
)
def _sc_gather(rows_hbm, p1_hbm, p2_hbm, r1_hbm, r2_hbm,
               p1v, p2v, r1v, r2v, sem):
    wid = lax.axis_index("s") * _NC + lax.axis_index("c")
    base = wid * TOK_W
    pltpu.sync_copy(p1_hbm.at[pl.ds(base, TOK_W)], p1v)
    pltpu.sync_copy(p2_hbm.at[pl.ds(base, TOK_W)], p2v)
    pltpu.async_copy(rows_hbm.at[p1v], r1v, sem).wait()
    pltpu.async_copy(rows_hbm.at[p2v], r2v, sem).wait()
    pltpu.sync_copy(r1v, r1_hbm.at[pl.ds(base, TOK_W), :])
    pltpu.sync_copy(r2v, r2_hbm.at[pl.ds(base, TOK_W), :])


def _combine_body(g1_ref, g2_ref, r1_ref, r2_ref, out_ref):
    out = g1_ref[...] * r1_ref[...] + g2_ref[...] * r2_ref[...]
    out_ref[...] = out[:, :D_OUT]


@jax.jit
def _combine(g1, g2, r1, r2):
    return pl.pallas_call(
        _combine_body,
        grid=(1,),
        in_specs=[
            pl.BlockSpec((B, 1), lambda i: (0, 0)),
            pl.BlockSpec((B, 1), lambda i: (0, 0)),
            pl.BlockSpec((B, D_OUT_PAD), lambda i: (0, 0)),
            pl.BlockSpec((B, D_OUT_PAD), lambda i: (0, 0)),
        ],
        out_specs=pl.BlockSpec((B, D_OUT), lambda i: (0, 0)),
        out_shape=jax.ShapeDtypeStruct((B, D_OUT), jnp.float32),
    )(g1, g2, r1, r2)


def kernel(x_num, Wg, bg, W1, b1, W2, b2):
    x = jnp.reshape(x_num, (x_num.shape[0], -1))
    p1, p2, g1, g2, be = _router(x, Wg, bg.reshape(1, N_EXPERTS))
    p1f, p2f = p1.reshape(B), p2.reshape(B)
    gx = jnp.zeros((P, D_IN), jnp.float32).at[p1f].set(x).at[p2f].set(x)
    W2p = jnp.pad(W2, ((0, 0), (0, 0), (0, D_OUT_PAD - D_OUT)))
    b2p = jnp.pad(b2, ((0, 0), (0, D_OUT_PAD - D_OUT)))
    rows = _grouped_gemm(be.reshape(64), gx, W1,
                         b1.reshape(N_EXPERTS, 1, D_BLOCK), W2p,
                         b2p.reshape(N_EXPERTS, 1, D_OUT_PAD))
    r1, r2 = rows[p1f], rows[p2f]
    return _combine(g1, g2, r1, r2)


# speedup vs baseline: 1.7609x; 1.7609x over previous
import functools

import jax
import jax.numpy as jnp
from jax import lax
from jax.experimental import pallas as pl
from jax.experimental.pallas import tpu as pltpu
from jax.experimental.pallas import tpu_sc as plsc

B = 4096
D_IN = 256
D_BLOCK = 512
N_EXPERTS = 8
TOP_K = 2
D_OUT = 10
D_OUT_PAD = 128

BT_G = 512
P = 2 * B + N_EXPERTS * BT_G
NBLK = P // BT_G
CH = 256


def _router_body(x_ref, wg_ref, bg_ref, p1_ref, p2_ref, g1_ref, g2_ref,
                 be_ref, rk_ref):
    x = x_ref[...]
    logits = jnp.dot(x, wg_ref[...], preferred_element_type=jnp.float32)
    logits = logits + bg_ref[...]

    eids = lax.broadcasted_iota(jnp.int32, logits.shape, 1)
    m1 = jnp.max(logits, axis=-1, keepdims=True)
    i1 = jnp.min(jnp.where(logits == m1, eids, N_EXPERTS), axis=-1,
                 keepdims=True)
    masked = jnp.where(eids == i1, jnp.float32(-jnp.inf), logits)
    m2 = jnp.max(masked, axis=-1, keepdims=True)
    i2 = jnp.min(jnp.where(masked == m2, eids, N_EXPERTS), axis=-1,
                 keepdims=True)
    e2 = jnp.exp(m2 - m1)
    g1 = 1.0 / (1.0 + e2)
    g2 = e2 / (1.0 + e2)
    ones16 = jnp.ones((1, 16), dtype=jnp.float32)
    g1_ref[...] = g1 * ones16
    g2_ref[...] = g2 * ones16

    rows = lax.broadcasted_iota(jnp.int32, (CH, CH), 0)
    cols = lax.broadcasted_iota(jnp.int32, (CH, CH), 1)
    ls = (cols < rows).astype(jnp.float32)
    eids16 = lax.broadcasted_iota(jnp.int32, (CH, 16), 1)
    e16 = jnp.where(eids16 < N_EXPERTS, eids16, eids16 - N_EXPERTS)

    running = jnp.zeros((1, 16), dtype=jnp.float32)
    for c in range(B // CH):
        sl = slice(c * CH, (c + 1) * CH)
        isel = jnp.where(eids16 < N_EXPERTS, i1[sl, :], i2[sl, :])
        oh = (isel == e16).astype(jnp.float32)
        ex = jnp.dot(ls, oh, preferred_element_type=jnp.float32)
        rk_ref[sl, :] = ex + running
        running = running + jnp.sum(oh, axis=0, keepdims=True)

    counts1 = running[:, :N_EXPERTS]
    counts = counts1 + running[:, N_EXPERTS:]

    pc = jnp.floor((counts + (BT_G - 1)) * (1.0 / BT_G)) * BT_G
    su = (lax.broadcasted_iota(jnp.int32, (N_EXPERTS, N_EXPERTS), 0)
          < lax.broadcasted_iota(jnp.int32, (N_EXPERTS, N_EXPERTS), 1))
    off = jnp.dot(pc, su.astype(jnp.float32),
                  preferred_element_type=jnp.float32)
    off2 = off + counts1

    for c in range(B // CH):
        sl = slice(c * CH, (c + 1) * CH)
        oh1 = (i1[sl, :] == e16[:, :N_EXPERTS]).astype(jnp.float32)
        oh2 = (i2[sl, :] == e16[:, :N_EXPERTS]).astype(jnp.float32)
        pos1 = jnp.sum(oh1 * (rk_ref[sl, :N_EXPERTS] + off), axis=-1,
                       keepdims=True)
        pos2 = jnp.sum(oh2 * (rk_ref[sl, N_EXPERTS:] + off2), axis=-1,
                       keepdims=True)
        p1_ref[sl, :] = pos1.astype(jnp.int32)
        p2_ref[sl, :] = pos2.astype(jnp.int32)

    bidx = lax.broadcasted_iota(jnp.int32, (64, N_EXPERTS), 0).astype(
        jnp.float32) * BT_G
    ends = off + pc
    be = jnp.sum((bidx >= ends).astype(jnp.float32), axis=-1, keepdims=True)
    be_ref[...] = jnp.minimum(be, N_EXPERTS - 1).astype(jnp.int32)


@jax.jit
def _router(x, Wg, bg2d):
    return pl.pallas_call(
        _router_body,
        grid=(1,),
        in_specs=[
            pl.BlockSpec((B, D_IN), lambda i: (0, 0)),
            pl.BlockSpec((D_IN, N_EXPERTS), lambda i: (0, 0)),
            pl.BlockSpec((1, N_EXPERTS), lambda i: (0, 0)),
        ],
        out_specs=[
            pl.BlockSpec((B, 1), lambda i: (0, 0)),
            pl.BlockSpec((B, 1), lambda i: (0, 0)),
            pl.BlockSpec((B, 16), lambda i: (0, 0)),
            pl.BlockSpec((B, 16), lambda i: (0, 0)),
            pl.BlockSpec((64, 1), lambda i: (0, 0)),
        ],
        out_shape=[
            jax.ShapeDtypeStruct((B, 1), jnp.int32),
            jax.ShapeDtypeStruct((B, 1), jnp.int32),
            jax.ShapeDtypeStruct((B, 16), jnp.float32),
            jax.ShapeDtypeStruct((B, 16), jnp.float32),
            jax.ShapeDtypeStruct((64, 1), jnp.int32),
        ],
        scratch_shapes=[
            pltpu.VMEM((B, 16), jnp.float32),
        ],
    )(x, Wg, bg2d)


_SC_INFO = plsc.get_sparse_core_info()
_NC, _NS = _SC_INFO.num_cores, _SC_INFO.num_subcores
_NW = _NC * _NS
TOK_W = B // _NW

_sc_mesh = plsc.VectorSubcoreMesh(core_axis_name="c", subcore_axis_name="s")


@functools.partial(
    pl.kernel, mesh=_sc_mesh,
    out_type=jax.ShapeDtypeStruct((P, D_IN), jnp.float32),
    scratch_types=[
        pltpu.VMEM((TOK_W, D_IN), jnp.float32),
        pltpu.VMEM((TOK_W,), jnp.int32),
        pltpu.VMEM((TOK_W,), jnp.int32),
        pltpu.SemaphoreType.DMA,
    ],
)
def _sc_scatter(x_hbm, p1_hbm, p2_hbm, gx_hbm, xv, p1v, p2v, sem):
    wid = lax.axis_index("s") * _NC + lax.axis_index("c")
    base = wid * TOK_W
    pltpu.sync_copy(x_hbm.at[pl.ds(base, TOK_W), :], xv)
    pltpu.sync_copy(p1_hbm.at[pl.ds(base, TOK_W)], p1v)
    pltpu.sync_copy(p2_hbm.at[pl.ds(base, TOK_W)], p2v)
    pltpu.async_copy(xv, gx_hbm.at[p1v], sem).wait()
    pltpu.async_copy(xv, gx_hbm.at[p2v], sem).wait()


@functools.partial(
    pl.kernel, mesh=_sc_mesh,
    out_type=jax.ShapeDtypeStruct((B, 16), jnp.float32),
    scratch_types=[
        pltpu.VMEM((TOK_W,), jnp.int32),
        pltpu.VMEM((TOK_W,), jnp.int32),
        pltpu.VMEM((TOK_W, D_OUT_PAD), jnp.float32),
        pltpu.VMEM((TOK_W, D_OUT_PAD), jnp.float32),
        pltpu.VMEM((TOK_W, 16), jnp.float32),
        pltpu.VMEM((TOK_W, 16), jnp.float32),
        pltpu.VMEM((TOK_W, 16), jnp.float32),
        pltpu.SemaphoreType.DMA,
    ],
)
def _sc_gather_combine(rows_hbm, p1_hbm, p2_hbm, g1_hbm, g2_hbm, out_hbm,
                       p1v, p2v, r1v, r2v, g1v, g2v, outv, sem):
    wid = lax.axis_index("s") * _NC + lax.axis_index("c")
    base = wid * TOK_W
    pltpu.sync_copy(p1_hbm.at[pl.ds(base, TOK_W)], p1v)
    pltpu.sync_copy(p2_hbm.at[pl.ds(base, TOK_W)], p2v)
    pltpu.sync_copy(g1_hbm.at[pl.ds(base, TOK_W), :], g1v)
    pltpu.sync_copy(g2_hbm.at[pl.ds(base, TOK_W), :], g2v)
    cp1 = pltpu.async_copy(rows_hbm.at[p1v], r1v, sem)
    cp2 = pltpu.async_copy(rows_hbm.at[p2v], r2v, sem)
    cp1.wait()
    cp2.wait()
    for i in range(TOK_W):
        outv[i, :] = (r1v[i, 0:16] * g1v[i, :] + r2v[i, 0:16] * g2v[i, :])
    pltpu.sync_copy(outv, out_hbm.at[pl.ds(base, TOK_W), :])


def _gemm_body(be_ref, gx_ref, w1_ref, b1_ref, w2_ref, b2_ref, rows_ref):
    x = gx_ref[...]
    h = jnp.dot(x, w1_ref[0], preferred_element_type=jnp.float32)
    h = jnp.maximum(h + b1_ref[0], 0.0)
    o = jnp.dot(h, w2_ref[0], preferred_element_type=jnp.float32)
    rows_ref[...] = o + b2_ref[0]


@jax.jit
def _grouped_gemm(be, gx, W1, b1, W2p, b2p):
    grid_spec = pltpu.PrefetchScalarGridSpec(
        num_scalar_prefetch=1,
        grid=(NBLK,),
        in_specs=[
            pl.BlockSpec((BT_G, D_IN), lambda i, be: (i, 0)),
            pl.BlockSpec((1, D_IN, D_BLOCK), lambda i, be: (be[i], 0, 0)),
            pl.BlockSpec((1, 1, D_BLOCK), lambda i, be: (be[i], 0, 0)),
            pl.BlockSpec((1, D_BLOCK, D_OUT_PAD), lambda i, be: (be[i], 0, 0)),
            pl.BlockSpec((1, 1, D_OUT_PAD), lambda i, be: (be[i], 0, 0)),
        ],
        out_specs=pl.BlockSpec((BT_G, D_OUT_PAD), lambda i, be: (i, 0)),
    )
    return pl.pallas_call(
        _gemm_body,
        grid_spec=grid_spec,
        out_shape=jax.ShapeDtypeStruct((P, D_OUT_PAD), jnp.float32),
    )(be, gx, W1, b1, W2p, b2p)


def kernel(x_num, Wg, bg, W1, b1, W2, b2):
    x = jnp.reshape(x_num, (x_num.shape[0], -1))
    p1, p2, g1b, g2b, be = _router(x, Wg, bg.reshape(1, N_EXPERTS))
    p1f, p2f = p1.reshape(B), p2.reshape(B)
    gx = _sc_scatter(x, p1f, p2f)
    W2p = jnp.pad(W2, ((0, 0), (0, 0), (0, D_OUT_PAD - D_OUT)))
    b2p = jnp.pad(b2, ((0, 0), (0, D_OUT_PAD - D_OUT)))
    rows = _grouped_gemm(be.reshape(64), gx, W1,
                         b1.reshape(N_EXPERTS, 1, D_BLOCK), W2p,
                         b2p.reshape(N_EXPERTS, 1, D_OUT_PAD))
    out16 = _sc_gather_combine(rows, p1f, p2f, g1b, g2b)
    return out16[:, :D_OUT]
